# baseline (speedup 1.0000x reference)
import functools

import jax
import jax.numpy as jnp
from jax import lax
from jax.experimental import pallas as pl
from jax.experimental.pallas import tpu as pltpu
from jax.experimental.pallas import tpu_sc as plsc

N = 10000
E = 320000
H = 128

NW = 32
CH = 80
UNIT = CH * NW
EA = 84 * UNIT
EB = 41 * UNIT
NCA = EA // CH
NCB = EB // CH
SPWA = NCA // 16
SPWB = NCB // 16
NP = 10240
NPS = NP // 16

BN = 1000
BE = 4000

_f32 = jnp.float32


_bf16 = jnp.bfloat16
_i16 = jnp.int16
_i32 = jnp.int32


def _enc_body(x_ref, w1_ref, b1_ref, w2_ref, b2_ref, o_ref):
    t = jnp.dot(x_ref[...], w1_ref[...], preferred_element_type=_f32) + b1_ref[...]
    t = jnp.maximum(t, 0.0)
    y = jnp.dot(t, w2_ref[...], preferred_element_type=_f32) + b2_ref[...]
    o_ref[...] = y


def _encoder(x, enc):
    W1, b1, W2, b2 = enc
    n, d = x.shape
    grid = n // BN
    return pl.pallas_call(
        _enc_body,
        grid=(grid,),
        in_specs=[
            pl.BlockSpec((BN, d), lambda i: (i, 0)),
            pl.BlockSpec((d, H), lambda i: (0, 0)),
            pl.BlockSpec((1, H), lambda i: (0, 0)),
            pl.BlockSpec((H, H), lambda i: (0, 0)),
            pl.BlockSpec((1, H), lambda i: (0, 0)),
        ],
        out_specs=pl.BlockSpec((BN, H), lambda i: (i, 0)),
        out_shape=jax.ShapeDtypeStruct((n, H), _f32),
    )(x, W1, b1.reshape(1, H), W2, b2.reshape(1, H))


def _edge_body(xi_ref, xj_ref, ea_ref,
               w1i_ref, w1j_ref, w1e_ref, b1_ref, w2_ref, b2_ref,
               mw1_ref, mb1_ref, mw2_ref, mb2_ref,
               o_ref):
    t = (jnp.dot(xi_ref[...].astype(_bf16), w1i_ref[...], preferred_element_type=_f32)
         + jnp.dot(xj_ref[...].astype(_bf16), w1j_ref[...], preferred_element_type=_f32)
         + jnp.dot(ea_ref[...].astype(_bf16), w1e_ref[...], preferred_element_type=_f32)
         + b1_ref[...])
    t = jnp.maximum(t, 0.0).astype(_bf16)
    emb = jnp.dot(t, w2_ref[...], preferred_element_type=_f32) + b2_ref[...]
    t = jnp.maximum(
        jnp.dot(emb.astype(_bf16), mw1_ref[...], preferred_element_type=_f32)
        + mb1_ref[...], 0.0).astype(_bf16)
    o_ref[...] = jnp.dot(t, mw2_ref[...], preferred_element_type=_f32) + mb2_ref[...]


def _edge_mlp(xi, xj, ea, et, msg):
    W1, b1, W2, b2 = et
    MW1, mb1, MW2, mb2 = msg
    ed = ea.shape[1]
    e = xi.shape[0]
    be = UNIT
    grid = e // UNIT
    full = lambda i: (0, 0)
    return pl.pallas_call(
        _edge_body,
        grid=(grid,),
        in_specs=[
            pl.BlockSpec((be, H), lambda i: (i, 0)),
            pl.BlockSpec((be, H), lambda i: (i, 0)),
            pl.BlockSpec((be, ed), lambda i: (i, 0)),
            pl.BlockSpec((H, H), full),
            pl.BlockSpec((H, H), full),
            pl.BlockSpec((ed, H), full),
            pl.BlockSpec((1, H), full),
            pl.BlockSpec((H, H), full),
            pl.BlockSpec((1, H), full),
            pl.BlockSpec((H, H), full),
            pl.BlockSpec((1, H), full),
            pl.BlockSpec((H, H), full),
            pl.BlockSpec((1, H), full),
        ],
        out_specs=pl.BlockSpec((be, H), lambda i: (i, 0)),
        out_shape=jax.ShapeDtypeStruct((e, H), _f32),
    )(xi, xj, ea,
      W1[:H].astype(_bf16), W1[H:2 * H].astype(_bf16), W1[2 * H:].astype(_bf16),
      b1.reshape(1, H), W2.astype(_bf16), b2.reshape(1, H),
      MW1.astype(_bf16), mb1.reshape(1, H), MW2.astype(_bf16), mb2.reshape(1, H))


def _sc_mesh():
    return plsc.VectorSubcoreMesh(core_axis_name="c", subcore_axis_name="s")


def _make_gather(eh, cpw):
    def body(h_hbm, src3_hbm, dst3_hbm, xi_hbm, xj_hbm,
             idx_d, idx_s, rows_a, rows_b, sem_a, sem_b):
        cid = lax.axis_index("c")
        sid = lax.axis_index("s")
        wid = cid * 16 + sid
        c0 = wid * cpw
        pltpu.sync_copy(dst3_hbm.at[pl.ds(c0, cpw)], idx_d)
        pltpu.sync_copy(src3_hbm.at[pl.ds(c0, cpw)], idx_s)

        def one_pass(idx_v, out_hbm):
            pltpu.async_copy(h_hbm.at[idx_v.at[0, 0]], rows_a, sem_a)

            def pair(j, carry):
                ce = 2 * j
                pltpu.make_async_copy(h_hbm.at[idx_v.at[ce, 0]], rows_a,
                                      sem_a).wait()
                pltpu.async_copy(h_hbm.at[idx_v.at[ce + 1, 0]], rows_b, sem_b)
                pltpu.sync_copy(rows_a, out_hbm.at[pl.ds((c0 + ce) * CH, CH)])
                pltpu.make_async_copy(h_hbm.at[idx_v.at[ce + 1, 0]], rows_b,
                                      sem_b).wait()

                @pl.when(ce + 2 < cpw)
                def _():
                    pltpu.async_copy(h_hbm.at[idx_v.at[ce + 2, 0]], rows_a, sem_a)

                pltpu.sync_copy(rows_b, out_hbm.at[pl.ds((c0 + ce + 1) * CH, CH)])
                return carry

            lax.fori_loop(0, cpw // 2, pair, 0)
            if cpw % 2 == 1:
                pltpu.make_async_copy(h_hbm.at[idx_v.at[cpw - 1, 0]], rows_a,
                                      sem_a).wait()
                pltpu.sync_copy(rows_a, out_hbm.at[pl.ds((c0 + cpw - 1) * CH, CH)])

        one_pass(idx_d, xi_hbm)
        one_pass(idx_s, xj_hbm)

    return pl.kernel(
        body, mesh=_sc_mesh(),
        out_type=(jax.ShapeDtypeStruct((eh, H), _f32),
                  jax.ShapeDtypeStruct((eh, H), _f32)),
        scratch_types=[
            pltpu.VMEM((cpw, 1, CH), jnp.int32),
            pltpu.VMEM((cpw, 1, CH), jnp.int32),
            pltpu.VMEM((CH, H), _f32),
            pltpu.VMEM((CH, H), _f32),
            pltpu.SemaphoreType.DMA,
            pltpu.SemaphoreType.DMA,
        ],
    )


_GATHER_A = _make_gather(EA, NCA // NW)
_GATHER_B = _make_gather(EB, NCB // NW)


def _make_scatter(eh, spw):
    def body(msg_hbm, dst3_hbm, z_hbm, p0_hbm, p1_hbm,
             idx_all, rows_a, rows_b, acc_s, sem_a, sem_b):
        cid = lax.axis_index("c")
        sid = lax.axis_index("s")
        wid = cid * 16 + sid
        r0 = sid * NPS
        pltpu.sync_copy(z_hbm.at[pl.ds(r0, NPS)], acc_s.at[pl.ds(r0, NPS)])
        c0 = wid * spw
        pltpu.sync_copy(dst3_hbm.at[pl.ds(c0, spw)], idx_all)
        plsc.subcore_barrier()

        pltpu.async_copy(msg_hbm.at[pl.ds(c0 * CH, CH)], rows_a, sem_a)

        def pair(j, carry):
            ce = 2 * j
            pltpu.make_async_copy(msg_hbm.at[pl.ds((c0 + ce) * CH, CH)], rows_a,
                                  sem_a).wait()
            pltpu.async_copy(msg_hbm.at[pl.ds((c0 + ce + 1) * CH, CH)], rows_b,
                             sem_b)
            pltpu.sync_copy(rows_a, acc_s.at[idx_all.at[ce, 0]], add=True)
            pltpu.make_async_copy(msg_hbm.at[pl.ds((c0 + ce + 1) * CH, CH)],
                                  rows_b, sem_b).wait()

            @pl.when(ce + 2 < spw)
            def _():
                pltpu.async_copy(msg_hbm.at[pl.ds((c0 + ce + 2) * CH, CH)],
                                 rows_a, sem_a)

            pltpu.sync_copy(rows_b, acc_s.at[idx_all.at[ce + 1, 0]], add=True)
            return carry

        lax.fori_loop(0, spw // 2, pair, 0)
        if spw % 2 == 1:
            pltpu.make_async_copy(msg_hbm.at[pl.ds((c0 + spw - 1) * CH, CH)],
                                  rows_a, sem_a).wait()
            pltpu.sync_copy(rows_a, acc_s.at[idx_all.at[spw - 1, 0]], add=True)
        plsc.subcore_barrier()

        @pl.when(cid == 0)
        def _():
            pltpu.sync_copy(acc_s.at[pl.ds(r0, NPS)], p0_hbm.at[pl.ds(r0, NPS)])

        @pl.when(cid == 1)
        def _():
            pltpu.sync_copy(acc_s.at[pl.ds(r0, NPS)], p1_hbm.at[pl.ds(r0, NPS)])

    return pl.kernel(
        body, mesh=_sc_mesh(),
        out_type=(jax.ShapeDtypeStruct((NP, H), _f32),
                  jax.ShapeDtypeStruct((NP, H), _f32)),
        scratch_types=[
            pltpu.VMEM((spw, 1, CH), jnp.int32),
            pltpu.VMEM((CH, H), _f32),
            pltpu.VMEM((CH, H), _f32),
            pltpu.VMEM_SHARED((NP, H), _f32),
            pltpu.SemaphoreType.DMA,
            pltpu.SemaphoreType.DMA,
        ],
    )


_SCATTER_A = _make_scatter(EA, NCA // NW)
_SCATTER_B = _make_scatter(EB, NCB // NW)


def _node_body(h_ref, a0_ref, a1_ref, a2_ref, a3_ref, w1h_ref, w1a_ref, b1_ref,
               w2_ref, b2_ref, g_ref, bln_ref, o_ref):
    h = h_ref[...]
    aggr = (a0_ref[...] + a1_ref[...]) + (a2_ref[...] + a3_ref[...])
    t = (jnp.dot(h, w1h_ref[...], preferred_element_type=_f32)
         + jnp.dot(aggr, w1a_ref[...], preferred_element_type=_f32)
         + b1_ref[...])
    t = jnp.maximum(t, 0.0)
    upd = jnp.dot(t, w2_ref[...], preferred_element_type=_f32) + b2_ref[...]
    y = h + upd
    m = jnp.mean(y, axis=1, keepdims=True)
    c = y - m
    v = jnp.mean(c * c, axis=1, keepdims=True)
    o_ref[...] = c * jax.lax.rsqrt(v + 1e-5) * g_ref[...] + bln_ref[...]


def _node_update(h, parts, upd, g, bln):
    W1, b1, W2, b2 = upd
    grid = N // BN
    full = lambda i: (0, 0)
    return pl.pallas_call(
        _node_body,
        grid=(grid,),
        in_specs=[
            pl.BlockSpec((BN, H), lambda i: (i, 0)),
            pl.BlockSpec((BN, H), lambda i: (i, 0)),
            pl.BlockSpec((BN, H), lambda i: (i, 0)),
            pl.BlockSpec((BN, H), lambda i: (i, 0)),
            pl.BlockSpec((BN, H), lambda i: (i, 0)),
            pl.BlockSpec((H, H), full),
            pl.BlockSpec((H, H), full),
            pl.BlockSpec((1, H), full),
            pl.BlockSpec((H, H), full),
            pl.BlockSpec((1, H), full),
            pl.BlockSpec((1, H), full),
            pl.BlockSpec((1, H), full),
        ],
        out_specs=pl.BlockSpec((BN, H), lambda i: (i, 0)),
        out_shape=jax.ShapeDtypeStruct((N, H), _f32),
    )(h, parts[0], parts[1], parts[2], parts[3], W1[:H], W1[H:],
      b1.reshape(1, H), W2, b2.reshape(1, H), g.reshape(1, H),
      bln.reshape(1, H))


def _gru_body(h_ref, hp_ref, wih_ref, bih_ref, whh_ref, bhh_ref, o_ref):
    gi = jnp.dot(h_ref[...], wih_ref[...], preferred_element_type=_f32) + bih_ref[...]
    gh = jnp.dot(hp_ref[...], whh_ref[...], preferred_element_type=_f32) + bhh_ref[...]
    i_r, i_z, i_n = gi[:, :H], gi[:, H:2 * H], gi[:, 2 * H:]
    h_r, h_z, h_n = gh[:, :H], gh[:, H:2 * H], gh[:, 2 * H:]
    r = jax.nn.sigmoid(i_r + h_r)
    z = jax.nn.sigmoid(i_z + h_z)
    n_gate = jnp.tanh(i_n + r * h_n)
    o_ref[...] = (1.0 - z) * n_gate + z * hp_ref[...]


def _gru(h, h_prev, gru):
    W_ih, b_ih, W_hh, b_hh = gru
    grid = N // BN
    full = lambda i: (0, 0)
    return pl.pallas_call(
        _gru_body,
        grid=(grid,),
        in_specs=[
            pl.BlockSpec((BN, H), lambda i: (i, 0)),
            pl.BlockSpec((BN, H), lambda i: (i, 0)),
            pl.BlockSpec((H, 3 * H), full),
            pl.BlockSpec((1, 3 * H), full),
            pl.BlockSpec((H, 3 * H), full),
            pl.BlockSpec((1, 3 * H), full),
        ],
        out_specs=pl.BlockSpec((BN, H), lambda i: (i, 0)),
        out_shape=jax.ShapeDtypeStruct((N, H), _f32),
    )(h, h_prev, W_ih, b_ih.reshape(1, 3 * H), W_hh, b_hh.reshape(1, 3 * H))


@jax.jit
def _run(x, edge_index, edge_attr, h_prev, params):
    src = edge_index[0]
    dst = edge_index[1]
    srcA3 = src[:EA].reshape(NCA, 1, CH)
    srcB3 = src[EA:].reshape(NCB, 1, CH)
    dstA3 = dst[:EA].reshape(NCA, 1, CH)
    dstB3 = dst[EA:].reshape(NCB, 1, CH)
    eaA = edge_attr[:EA]
    eaB = edge_attr[EA:]
    zeros = jnp.zeros((NP, H), _f32)
    h = _encoder(x, params['enc'])
    g, bln = params['ln']
    for lp in params['layers']:
        xiA, xjA = _GATHER_A(h, srcA3, dstA3)
        xiB, xjB = _GATHER_B(h, srcB3, dstB3)
        msgA = _edge_mlp(xiA, xjA, eaA, lp['et'], lp['msg'])
        msgB = _edge_mlp(xiB, xjB, eaB, lp['et'], lp['msg'])
        pA0, pA1 = _SCATTER_A(msgA, dstA3, zeros)
        pB0, pB1 = _SCATTER_B(msgB, dstB3, zeros)
        h = _node_update(h, (pA0, pA1, pB0, pB1), lp['upd'], g, bln)
    return _gru(h, h_prev, params['gru'])


def kernel(x, edge_index, edge_attr, h_prev, params):
    return _run(x, edge_index, edge_attr, h_prev, params)


# speedup vs baseline: 1.0188x; 1.0188x over previous
import functools

import jax
import jax.numpy as jnp
from jax import lax
from jax.experimental import pallas as pl
from jax.experimental.pallas import tpu as pltpu
from jax.experimental.pallas import tpu_sc as plsc

N = 10000
E = 320000
H = 128

NW = 32
CH = 80
UNIT = CH * NW
EA = 62 * UNIT
EB = 63 * UNIT
NCA = EA // CH
NCB = EB // CH
SPWA = NCA // 16
SPWB = NCB // 16
NP = 10240
NPS = NP // 16

BN = 1000
BE = 4000

_f32 = jnp.float32


_bf16 = jnp.bfloat16
_i16 = jnp.int16
_i32 = jnp.int32


def _enc_body(x_ref, w1_ref, b1_ref, w2_ref, b2_ref, o_ref):
    t = jnp.dot(x_ref[...], w1_ref[...], preferred_element_type=_f32) + b1_ref[...]
    t = jnp.maximum(t, 0.0)
    y = jnp.dot(t, w2_ref[...], preferred_element_type=_f32) + b2_ref[...]
    o_ref[...] = y


def _encoder(x, enc):
    W1, b1, W2, b2 = enc
    n, d = x.shape
    grid = n // BN
    return pl.pallas_call(
        _enc_body,
        grid=(grid,),
        in_specs=[
            pl.BlockSpec((BN, d), lambda i: (i, 0)),
            pl.BlockSpec((d, H), lambda i: (0, 0)),
            pl.BlockSpec((1, H), lambda i: (0, 0)),
            pl.BlockSpec((H, H), lambda i: (0, 0)),
            pl.BlockSpec((1, H), lambda i: (0, 0)),
        ],
        out_specs=pl.BlockSpec((BN, H), lambda i: (i, 0)),
        out_shape=jax.ShapeDtypeStruct((n, H), _f32),
    )(x, W1, b1.reshape(1, H), W2, b2.reshape(1, H))


def _edge_body(xi_ref, xj_ref, ea_ref,
               w1i_ref, w1j_ref, w1e_ref, b1_ref, w2_ref, b2_ref,
               mw1_ref, mb1_ref, mw2_ref, mb2_ref,
               o_ref):
    t = (jnp.dot(xi_ref[...].astype(_bf16), w1i_ref[...], preferred_element_type=_f32)
         + jnp.dot(xj_ref[...].astype(_bf16), w1j_ref[...], preferred_element_type=_f32)
         + jnp.dot(ea_ref[...].astype(_bf16), w1e_ref[...], preferred_element_type=_f32)
         + b1_ref[...])
    t = jnp.maximum(t, 0.0).astype(_bf16)
    emb = jnp.dot(t, w2_ref[...], preferred_element_type=_f32) + b2_ref[...]
    t = jnp.maximum(
        jnp.dot(emb.astype(_bf16), mw1_ref[...], preferred_element_type=_f32)
        + mb1_ref[...], 0.0).astype(_bf16)
    o_ref[...] = jnp.dot(t, mw2_ref[...], preferred_element_type=_f32) + mb2_ref[...]


def _edge_mlp(xi, xj, ea, et, msg):
    W1, b1, W2, b2 = et
    MW1, mb1, MW2, mb2 = msg
    ed = ea.shape[1]
    e = xi.shape[0]
    be = UNIT
    grid = e // UNIT
    full = lambda i: (0, 0)
    return pl.pallas_call(
        _edge_body,
        grid=(grid,),
        in_specs=[
            pl.BlockSpec((be, H), lambda i: (i, 0)),
            pl.BlockSpec((be, H), lambda i: (i, 0)),
            pl.BlockSpec((be, ed), lambda i: (i, 0)),
            pl.BlockSpec((H, H), full),
            pl.BlockSpec((H, H), full),
            pl.BlockSpec((ed, H), full),
            pl.BlockSpec((1, H), full),
            pl.BlockSpec((H, H), full),
            pl.BlockSpec((1, H), full),
            pl.BlockSpec((H, H), full),
            pl.BlockSpec((1, H), full),
            pl.BlockSpec((H, H), full),
            pl.BlockSpec((1, H), full),
        ],
        out_specs=pl.BlockSpec((be, H), lambda i: (i, 0)),
        out_shape=jax.ShapeDtypeStruct((e, H), _f32),
    )(xi, xj, ea,
      W1[:H].astype(_bf16), W1[H:2 * H].astype(_bf16), W1[2 * H:].astype(_bf16),
      b1.reshape(1, H), W2.astype(_bf16), b2.reshape(1, H),
      MW1.astype(_bf16), mb1.reshape(1, H), MW2.astype(_bf16), mb2.reshape(1, H))


def _sc_mesh():
    return plsc.VectorSubcoreMesh(core_axis_name="c", subcore_axis_name="s")


def _make_gather(eh, cpw):
    def body(h_hbm, src3_hbm, dst3_hbm, xi_hbm, xj_hbm,
             idx_d, idx_s, rows_a, rows_b, sem_a, sem_b):
        cid = lax.axis_index("c")
        sid = lax.axis_index("s")
        wid = cid * 16 + sid
        c0 = wid * cpw
        pltpu.sync_copy(dst3_hbm.at[pl.ds(c0, cpw)], idx_d)
        pltpu.sync_copy(src3_hbm.at[pl.ds(c0, cpw)], idx_s)

        def one_pass(idx_v, out_hbm):
            pltpu.async_copy(h_hbm.at[idx_v.at[0, 0]], rows_a, sem_a)

            def pair(j, carry):
                ce = 2 * j
                pltpu.make_async_copy(h_hbm.at[idx_v.at[ce, 0]], rows_a,
                                      sem_a).wait()
                pltpu.async_copy(h_hbm.at[idx_v.at[ce + 1, 0]], rows_b, sem_b)
                pltpu.sync_copy(rows_a, out_hbm.at[pl.ds((c0 + ce) * CH, CH)])
                pltpu.make_async_copy(h_hbm.at[idx_v.at[ce + 1, 0]], rows_b,
                                      sem_b).wait()

                @pl.when(ce + 2 < cpw)
                def _():
                    pltpu.async_copy(h_hbm.at[idx_v.at[ce + 2, 0]], rows_a, sem_a)

                pltpu.sync_copy(rows_b, out_hbm.at[pl.ds((c0 + ce + 1) * CH, CH)])
                return carry

            lax.fori_loop(0, cpw // 2, pair, 0)
            if cpw % 2 == 1:
                pltpu.make_async_copy(h_hbm.at[idx_v.at[cpw - 1, 0]], rows_a,
                                      sem_a).wait()
                pltpu.sync_copy(rows_a, out_hbm.at[pl.ds((c0 + cpw - 1) * CH, CH)])

        one_pass(idx_d, xi_hbm)
        one_pass(idx_s, xj_hbm)

    return pl.kernel(
        body, mesh=_sc_mesh(),
        out_type=(jax.ShapeDtypeStruct((eh, H), _f32),
                  jax.ShapeDtypeStruct((eh, H), _f32)),
        scratch_types=[
            pltpu.VMEM((cpw, 1, CH), jnp.int32),
            pltpu.VMEM((cpw, 1, CH), jnp.int32),
            pltpu.VMEM((CH, H), _f32),
            pltpu.VMEM((CH, H), _f32),
            pltpu.SemaphoreType.DMA,
            pltpu.SemaphoreType.DMA,
        ],
    )


_GATHER_A = _make_gather(EA, NCA // NW)
_GATHER_B = _make_gather(EB, NCB // NW)


def _make_scatter(eh, spw):
    def body(msg_hbm, dst3_hbm, z_hbm, p0_hbm, p1_hbm,
             idx_all, rows_a, rows_b, acc_s, sem_a, sem_b):
        cid = lax.axis_index("c")
        sid = lax.axis_index("s")
        wid = cid * 16 + sid
        r0 = sid * NPS
        pltpu.sync_copy(z_hbm.at[pl.ds(r0, NPS)], acc_s.at[pl.ds(r0, NPS)])
        c0 = wid * spw
        pltpu.sync_copy(dst3_hbm.at[pl.ds(c0, spw)], idx_all)
        plsc.subcore_barrier()

        pltpu.async_copy(msg_hbm.at[pl.ds(c0 * CH, CH)], rows_a, sem_a)

        def pair(j, carry):
            ce = 2 * j
            pltpu.make_async_copy(msg_hbm.at[pl.ds((c0 + ce) * CH, CH)], rows_a,
                                  sem_a).wait()
            pltpu.async_copy(msg_hbm.at[pl.ds((c0 + ce + 1) * CH, CH)], rows_b,
                             sem_b)
            pltpu.sync_copy(rows_a, acc_s.at[idx_all.at[ce, 0]], add=True)
            pltpu.make_async_copy(msg_hbm.at[pl.ds((c0 + ce + 1) * CH, CH)],
                                  rows_b, sem_b).wait()

            @pl.when(ce + 2 < spw)
            def _():
                pltpu.async_copy(msg_hbm.at[pl.ds((c0 + ce + 2) * CH, CH)],
                                 rows_a, sem_a)

            pltpu.sync_copy(rows_b, acc_s.at[idx_all.at[ce + 1, 0]], add=True)
            return carry

        lax.fori_loop(0, spw // 2, pair, 0)
        if spw % 2 == 1:
            pltpu.make_async_copy(msg_hbm.at[pl.ds((c0 + spw - 1) * CH, CH)],
                                  rows_a, sem_a).wait()
            pltpu.sync_copy(rows_a, acc_s.at[idx_all.at[spw - 1, 0]], add=True)
        plsc.subcore_barrier()

        @pl.when(cid == 0)
        def _():
            pltpu.sync_copy(acc_s.at[pl.ds(r0, NPS)], p0_hbm.at[pl.ds(r0, NPS)])

        @pl.when(cid == 1)
        def _():
            pltpu.sync_copy(acc_s.at[pl.ds(r0, NPS)], p1_hbm.at[pl.ds(r0, NPS)])

    return pl.kernel(
        body, mesh=_sc_mesh(),
        out_type=(jax.ShapeDtypeStruct((NP, H), _f32),
                  jax.ShapeDtypeStruct((NP, H), _f32)),
        scratch_types=[
            pltpu.VMEM((spw, 1, CH), jnp.int32),
            pltpu.VMEM((CH, H), _f32),
            pltpu.VMEM((CH, H), _f32),
            pltpu.VMEM_SHARED((NP, H), _f32),
            pltpu.SemaphoreType.DMA,
            pltpu.SemaphoreType.DMA,
        ],
    )


_SCATTER_A = _make_scatter(EA, NCA // NW)
_SCATTER_B = _make_scatter(EB, NCB // NW)


def _node_body(h_ref, a0_ref, a1_ref, a2_ref, a3_ref, w1h_ref, w1a_ref, b1_ref,
               w2_ref, b2_ref, g_ref, bln_ref, o_ref):
    h = h_ref[...]
    aggr = (a0_ref[...] + a1_ref[...]) + (a2_ref[...] + a3_ref[...])
    t = (jnp.dot(h, w1h_ref[...], preferred_element_type=_f32)
         + jnp.dot(aggr, w1a_ref[...], preferred_element_type=_f32)
         + b1_ref[...])
    t = jnp.maximum(t, 0.0)
    upd = jnp.dot(t, w2_ref[...], preferred_element_type=_f32) + b2_ref[...]
    y = h + upd
    m = jnp.mean(y, axis=1, keepdims=True)
    c = y - m
    v = jnp.mean(c * c, axis=1, keepdims=True)
    o_ref[...] = c * jax.lax.rsqrt(v + 1e-5) * g_ref[...] + bln_ref[...]


def _node_update(h, parts, upd, g, bln):
    W1, b1, W2, b2 = upd
    grid = N // BN
    full = lambda i: (0, 0)
    return pl.pallas_call(
        _node_body,
        grid=(grid,),
        in_specs=[
            pl.BlockSpec((BN, H), lambda i: (i, 0)),
            pl.BlockSpec((BN, H), lambda i: (i, 0)),
            pl.BlockSpec((BN, H), lambda i: (i, 0)),
            pl.BlockSpec((BN, H), lambda i: (i, 0)),
            pl.BlockSpec((BN, H), lambda i: (i, 0)),
            pl.BlockSpec((H, H), full),
            pl.BlockSpec((H, H), full),
            pl.BlockSpec((1, H), full),
            pl.BlockSpec((H, H), full),
            pl.BlockSpec((1, H), full),
            pl.BlockSpec((1, H), full),
            pl.BlockSpec((1, H), full),
        ],
        out_specs=pl.BlockSpec((BN, H), lambda i: (i, 0)),
        out_shape=jax.ShapeDtypeStruct((N, H), _f32),
    )(h, parts[0], parts[1], parts[2], parts[3], W1[:H], W1[H:],
      b1.reshape(1, H), W2, b2.reshape(1, H), g.reshape(1, H),
      bln.reshape(1, H))


def _gru_body(h_ref, hp_ref, wih_ref, bih_ref, whh_ref, bhh_ref, o_ref):
    gi = jnp.dot(h_ref[...], wih_ref[...], preferred_element_type=_f32) + bih_ref[...]
    gh = jnp.dot(hp_ref[...], whh_ref[...], preferred_element_type=_f32) + bhh_ref[...]
    i_r, i_z, i_n = gi[:, :H], gi[:, H:2 * H], gi[:, 2 * H:]
    h_r, h_z, h_n = gh[:, :H], gh[:, H:2 * H], gh[:, 2 * H:]
    r = jax.nn.sigmoid(i_r + h_r)
    z = jax.nn.sigmoid(i_z + h_z)
    n_gate = jnp.tanh(i_n + r * h_n)
    o_ref[...] = (1.0 - z) * n_gate + z * hp_ref[...]


def _gru(h, h_prev, gru):
    W_ih, b_ih, W_hh, b_hh = gru
    grid = N // BN
    full = lambda i: (0, 0)
    return pl.pallas_call(
        _gru_body,
        grid=(grid,),
        in_specs=[
            pl.BlockSpec((BN, H), lambda i: (i, 0)),
            pl.BlockSpec((BN, H), lambda i: (i, 0)),
            pl.BlockSpec((H, 3 * H), full),
            pl.BlockSpec((1, 3 * H), full),
            pl.BlockSpec((H, 3 * H), full),
            pl.BlockSpec((1, 3 * H), full),
        ],
        out_specs=pl.BlockSpec((BN, H), lambda i: (i, 0)),
        out_shape=jax.ShapeDtypeStruct((N, H), _f32),
    )(h, h_prev, W_ih, b_ih.reshape(1, 3 * H), W_hh, b_hh.reshape(1, 3 * H))


@jax.jit
def _run(x, edge_index, edge_attr, h_prev, params):
    src = edge_index[0]
    dst = edge_index[1]
    srcA3 = src[:EA].reshape(NCA, 1, CH)
    srcB3 = src[EA:].reshape(NCB, 1, CH)
    dstA3 = dst[:EA].reshape(NCA, 1, CH)
    dstB3 = dst[EA:].reshape(NCB, 1, CH)
    eaA = edge_attr[:EA]
    eaB = edge_attr[EA:]
    zeros = jnp.zeros((NP, H), _f32)
    h = _encoder(x, params['enc'])
    g, bln = params['ln']
    for lp in params['layers']:
        xiA, xjA = _GATHER_A(h, srcA3, dstA3)
        xiB, xjB = _GATHER_B(h, srcB3, dstB3)
        msgA = _edge_mlp(xiA, xjA, eaA, lp['et'], lp['msg'])
        msgB = _edge_mlp(xiB, xjB, eaB, lp['et'], lp['msg'])
        pA0, pA1 = _SCATTER_A(msgA, dstA3, zeros)
        pB0, pB1 = _SCATTER_B(msgB, dstB3, zeros)
        h = _node_update(h, (pA0, pA1, pB0, pB1), lp['upd'], g, bln)
    return _gru(h, h_prev, params['gru'])


def kernel(x, edge_index, edge_attr, h_prev, params):
    return _run(x, edge_index, edge_attr, h_prev, params)


# speedup vs baseline: 1.0397x; 1.0206x over previous
import functools

import jax
import jax.numpy as jnp
from jax import lax
from jax.experimental import pallas as pl
from jax.experimental.pallas import tpu as pltpu
from jax.experimental.pallas import tpu_sc as plsc

N = 10000
E = 320000
H = 128

NW = 32
CH = 80
UNIT = CH * NW
EA = 62 * UNIT
EB = 63 * UNIT
NCA = EA // CH
NCB = EB // CH
SPWA = NCA // 16
SPWB = NCB // 16
NP = 10240
NPS = NP // 16

BN = 1000
BE = 4000

_f32 = jnp.float32


_bf16 = jnp.bfloat16
_i16 = jnp.int16
_i32 = jnp.int32


def _enc_body(x_ref, w1_ref, b1_ref, w2_ref, b2_ref, o_ref):
    t = jnp.dot(x_ref[...], w1_ref[...], preferred_element_type=_f32) + b1_ref[...]
    t = jnp.maximum(t, 0.0)
    y = jnp.dot(t, w2_ref[...], preferred_element_type=_f32) + b2_ref[...]
    o_ref[...] = y


def _encoder(x, enc):
    W1, b1, W2, b2 = enc
    n, d = x.shape
    grid = n // BN
    return pl.pallas_call(
        _enc_body,
        grid=(grid,),
        in_specs=[
            pl.BlockSpec((BN, d), lambda i: (i, 0)),
            pl.BlockSpec((d, H), lambda i: (0, 0)),
            pl.BlockSpec((1, H), lambda i: (0, 0)),
            pl.BlockSpec((H, H), lambda i: (0, 0)),
            pl.BlockSpec((1, H), lambda i: (0, 0)),
        ],
        out_specs=pl.BlockSpec((BN, H), lambda i: (i, 0)),
        out_shape=jax.ShapeDtypeStruct((n, H), _f32),
    )(x, W1, b1.reshape(1, H), W2, b2.reshape(1, H))


def _edge_body(xi_ref, xj_ref, ea_ref,
               w1i_ref, w1j_ref, w1e_ref, b1_ref, w2_ref, b2_ref,
               mw1_ref, mb1_ref, mw2_ref, mb2_ref,
               o_ref):
    t = (jnp.dot(xi_ref[...].astype(_bf16), w1i_ref[...], preferred_element_type=_f32)
         + jnp.dot(xj_ref[...].astype(_bf16), w1j_ref[...], preferred_element_type=_f32)
         + jnp.dot(ea_ref[...].astype(_bf16), w1e_ref[...], preferred_element_type=_f32)
         + b1_ref[...])
    t = jnp.maximum(t, 0.0).astype(_bf16)
    emb = jnp.dot(t, w2_ref[...], preferred_element_type=_f32) + b2_ref[...]
    t = jnp.maximum(
        jnp.dot(emb.astype(_bf16), mw1_ref[...], preferred_element_type=_f32)
        + mb1_ref[...], 0.0).astype(_bf16)
    o_ref[...] = jnp.dot(t, mw2_ref[...], preferred_element_type=_f32) + mb2_ref[...]


def _edge_mlp(xi, xj, ea, et, msg):
    W1, b1, W2, b2 = et
    MW1, mb1, MW2, mb2 = msg
    ed = ea.shape[1]
    e = xi.shape[0]
    be = e // 40
    grid = 40
    full = lambda i: (0, 0)
    return pl.pallas_call(
        _edge_body,
        grid=(grid,),
        in_specs=[
            pl.BlockSpec((be, H), lambda i: (i, 0)),
            pl.BlockSpec((be, H), lambda i: (i, 0)),
            pl.BlockSpec((be, ed), lambda i: (i, 0)),
            pl.BlockSpec((H, H), full),
            pl.BlockSpec((H, H), full),
            pl.BlockSpec((ed, H), full),
            pl.BlockSpec((1, H), full),
            pl.BlockSpec((H, H), full),
            pl.BlockSpec((1, H), full),
            pl.BlockSpec((H, H), full),
            pl.BlockSpec((1, H), full),
            pl.BlockSpec((H, H), full),
            pl.BlockSpec((1, H), full),
        ],
        out_specs=pl.BlockSpec((be, H), lambda i: (i, 0)),
        out_shape=jax.ShapeDtypeStruct((e, H), _f32),
    )(xi, xj, ea,
      W1[:H].astype(_bf16), W1[H:2 * H].astype(_bf16), W1[2 * H:].astype(_bf16),
      b1.reshape(1, H), W2.astype(_bf16), b2.reshape(1, H),
      MW1.astype(_bf16), mb1.reshape(1, H), MW2.astype(_bf16), mb2.reshape(1, H))


def _sc_mesh():
    return plsc.VectorSubcoreMesh(core_axis_name="c", subcore_axis_name="s")


def _make_gather(eh, cpw):
    def body(h_hbm, src3_hbm, dst3_hbm, xi_hbm, xj_hbm,
             idx_d, idx_s, rows_a, rows_b, sem_a, sem_b):
        cid = lax.axis_index("c")
        sid = lax.axis_index("s")
        wid = cid * 16 + sid
        c0 = wid * cpw
        pltpu.sync_copy(dst3_hbm.at[pl.ds(c0, cpw)], idx_d)
        pltpu.sync_copy(src3_hbm.at[pl.ds(c0, cpw)], idx_s)

        def one_pass(idx_v, out_hbm):
            pltpu.async_copy(h_hbm.at[idx_v.at[0, 0]], rows_a, sem_a)

            def pair(j, carry):
                ce = 2 * j
                pltpu.make_async_copy(h_hbm.at[idx_v.at[ce, 0]], rows_a,
                                      sem_a).wait()
                pltpu.async_copy(h_hbm.at[idx_v.at[ce + 1, 0]], rows_b, sem_b)
                pltpu.sync_copy(rows_a, out_hbm.at[pl.ds((c0 + ce) * CH, CH)])
                pltpu.make_async_copy(h_hbm.at[idx_v.at[ce + 1, 0]], rows_b,
                                      sem_b).wait()

                @pl.when(ce + 2 < cpw)
                def _():
                    pltpu.async_copy(h_hbm.at[idx_v.at[ce + 2, 0]], rows_a, sem_a)

                pltpu.sync_copy(rows_b, out_hbm.at[pl.ds((c0 + ce + 1) * CH, CH)])
                return carry

            lax.fori_loop(0, cpw // 2, pair, 0)
            if cpw % 2 == 1:
                pltpu.make_async_copy(h_hbm.at[idx_v.at[cpw - 1, 0]], rows_a,
                                      sem_a).wait()
                pltpu.sync_copy(rows_a, out_hbm.at[pl.ds((c0 + cpw - 1) * CH, CH)])

        one_pass(idx_d, xi_hbm)
        one_pass(idx_s, xj_hbm)

    return pl.kernel(
        body, mesh=_sc_mesh(),
        out_type=(jax.ShapeDtypeStruct((eh, H), _f32),
                  jax.ShapeDtypeStruct((eh, H), _f32)),
        scratch_types=[
            pltpu.VMEM((cpw, 1, CH), jnp.int32),
            pltpu.VMEM((cpw, 1, CH), jnp.int32),
            pltpu.VMEM((CH, H), _f32),
            pltpu.VMEM((CH, H), _f32),
            pltpu.SemaphoreType.DMA,
            pltpu.SemaphoreType.DMA,
        ],
    )


_GATHER_A = _make_gather(EA, NCA // NW)
_GATHER_B = _make_gather(EB, NCB // NW)


def _make_scatter(eh, spw):
    def body(msg_hbm, dst3_hbm, z_hbm, p0_hbm, p1_hbm,
             idx_all, rows_a, rows_b, acc_s, sem_a, sem_b):
        cid = lax.axis_index("c")
        sid = lax.axis_index("s")
        wid = cid * 16 + sid
        r0 = sid * NPS
        pltpu.sync_copy(z_hbm.at[pl.ds(r0, NPS)], acc_s.at[pl.ds(r0, NPS)])
        c0 = wid * spw
        pltpu.sync_copy(dst3_hbm.at[pl.ds(c0, spw)], idx_all)
        plsc.subcore_barrier()

        pltpu.async_copy(msg_hbm.at[pl.ds(c0 * CH, CH)], rows_a, sem_a)

        def pair(j, carry):
            ce = 2 * j
            pltpu.make_async_copy(msg_hbm.at[pl.ds((c0 + ce) * CH, CH)], rows_a,
                                  sem_a).wait()
            pltpu.async_copy(msg_hbm.at[pl.ds((c0 + ce + 1) * CH, CH)], rows_b,
                             sem_b)
            pltpu.sync_copy(rows_a, acc_s.at[idx_all.at[ce, 0]], add=True)
            pltpu.make_async_copy(msg_hbm.at[pl.ds((c0 + ce + 1) * CH, CH)],
                                  rows_b, sem_b).wait()

            @pl.when(ce + 2 < spw)
            def _():
                pltpu.async_copy(msg_hbm.at[pl.ds((c0 + ce + 2) * CH, CH)],
                                 rows_a, sem_a)

            pltpu.sync_copy(rows_b, acc_s.at[idx_all.at[ce + 1, 0]], add=True)
            return carry

        lax.fori_loop(0, spw // 2, pair, 0)
        if spw % 2 == 1:
            pltpu.make_async_copy(msg_hbm.at[pl.ds((c0 + spw - 1) * CH, CH)],
                                  rows_a, sem_a).wait()
            pltpu.sync_copy(rows_a, acc_s.at[idx_all.at[spw - 1, 0]], add=True)
        plsc.subcore_barrier()

        @pl.when(cid == 0)
        def _():
            pltpu.sync_copy(acc_s.at[pl.ds(r0, NPS)], p0_hbm.at[pl.ds(r0, NPS)])

        @pl.when(cid == 1)
        def _():
            pltpu.sync_copy(acc_s.at[pl.ds(r0, NPS)], p1_hbm.at[pl.ds(r0, NPS)])

    return pl.kernel(
        body, mesh=_sc_mesh(),
        out_type=(jax.ShapeDtypeStruct((NP, H), _f32),
                  jax.ShapeDtypeStruct((NP, H), _f32)),
        scratch_types=[
            pltpu.VMEM((spw, 1, CH), jnp.int32),
            pltpu.VMEM((CH, H), _f32),
            pltpu.VMEM((CH, H), _f32),
            pltpu.VMEM_SHARED((NP, H), _f32),
            pltpu.SemaphoreType.DMA,
            pltpu.SemaphoreType.DMA,
        ],
    )


_SCATTER_A = _make_scatter(EA, NCA // NW)
_SCATTER_B = _make_scatter(EB, NCB // NW)


def _node_body(h_ref, a0_ref, a1_ref, a2_ref, a3_ref, w1h_ref, w1a_ref, b1_ref,
               w2_ref, b2_ref, g_ref, bln_ref, o_ref):
    h = h_ref[...]
    aggr = (a0_ref[...] + a1_ref[...]) + (a2_ref[...] + a3_ref[...])
    t = (jnp.dot(h, w1h_ref[...], preferred_element_type=_f32)
         + jnp.dot(aggr, w1a_ref[...], preferred_element_type=_f32)
         + b1_ref[...])
    t = jnp.maximum(t, 0.0)
    upd = jnp.dot(t, w2_ref[...], preferred_element_type=_f32) + b2_ref[...]
    y = h + upd
    m = jnp.mean(y, axis=1, keepdims=True)
    c = y - m
    v = jnp.mean(c * c, axis=1, keepdims=True)
    o_ref[...] = c * jax.lax.rsqrt(v + 1e-5) * g_ref[...] + bln_ref[...]


def _node_update(h, parts, upd, g, bln):
    W1, b1, W2, b2 = upd
    grid = N // BN
    full = lambda i: (0, 0)
    return pl.pallas_call(
        _node_body,
        grid=(grid,),
        in_specs=[
            pl.BlockSpec((BN, H), lambda i: (i, 0)),
            pl.BlockSpec((BN, H), lambda i: (i, 0)),
            pl.BlockSpec((BN, H), lambda i: (i, 0)),
            pl.BlockSpec((BN, H), lambda i: (i, 0)),
            pl.BlockSpec((BN, H), lambda i: (i, 0)),
            pl.BlockSpec((H, H), full),
            pl.BlockSpec((H, H), full),
            pl.BlockSpec((1, H), full),
            pl.BlockSpec((H, H), full),
            pl.BlockSpec((1, H), full),
            pl.BlockSpec((1, H), full),
            pl.BlockSpec((1, H), full),
        ],
        out_specs=pl.BlockSpec((BN, H), lambda i: (i, 0)),
        out_shape=jax.ShapeDtypeStruct((N, H), _f32),
    )(h, parts[0], parts[1], parts[2], parts[3], W1[:H], W1[H:],
      b1.reshape(1, H), W2, b2.reshape(1, H), g.reshape(1, H),
      bln.reshape(1, H))


def _gru_body(h_ref, hp_ref, wih_ref, bih_ref, whh_ref, bhh_ref, o_ref):
    gi = jnp.dot(h_ref[...], wih_ref[...], preferred_element_type=_f32) + bih_ref[...]
    gh = jnp.dot(hp_ref[...], whh_ref[...], preferred_element_type=_f32) + bhh_ref[...]
    i_r, i_z, i_n = gi[:, :H], gi[:, H:2 * H], gi[:, 2 * H:]
    h_r, h_z, h_n = gh[:, :H], gh[:, H:2 * H], gh[:, 2 * H:]
    r = jax.nn.sigmoid(i_r + h_r)
    z = jax.nn.sigmoid(i_z + h_z)
    n_gate = jnp.tanh(i_n + r * h_n)
    o_ref[...] = (1.0 - z) * n_gate + z * hp_ref[...]


def _gru(h, h_prev, gru):
    W_ih, b_ih, W_hh, b_hh = gru
    grid = N // BN
    full = lambda i: (0, 0)
    return pl.pallas_call(
        _gru_body,
        grid=(grid,),
        in_specs=[
            pl.BlockSpec((BN, H), lambda i: (i, 0)),
            pl.BlockSpec((BN, H), lambda i: (i, 0)),
            pl.BlockSpec((H, 3 * H), full),
            pl.BlockSpec((1, 3 * H), full),
            pl.BlockSpec((H, 3 * H), full),
            pl.BlockSpec((1, 3 * H), full),
        ],
        out_specs=pl.BlockSpec((BN, H), lambda i: (i, 0)),
        out_shape=jax.ShapeDtypeStruct((N, H), _f32),
    )(h, h_prev, W_ih, b_ih.reshape(1, 3 * H), W_hh, b_hh.reshape(1, 3 * H))


@jax.jit
def _run(x, edge_index, edge_attr, h_prev, params):
    src = edge_index[0]
    dst = edge_index[1]
    srcA3 = src[:EA].reshape(NCA, 1, CH)
    srcB3 = src[EA:].reshape(NCB, 1, CH)
    dstA3 = dst[:EA].reshape(NCA, 1, CH)
    dstB3 = dst[EA:].reshape(NCB, 1, CH)
    eaA = edge_attr[:EA]
    eaB = edge_attr[EA:]
    zeros = jnp.zeros((NP, H), _f32)
    h = _encoder(x, params['enc'])
    g, bln = params['ln']
    for lp in params['layers']:
        xiA, xjA = _GATHER_A(h, srcA3, dstA3)
        xiB, xjB = _GATHER_B(h, srcB3, dstB3)
        msgA = _edge_mlp(xiA, xjA, eaA, lp['et'], lp['msg'])
        msgB = _edge_mlp(xiB, xjB, eaB, lp['et'], lp['msg'])
        pA0, pA1 = _SCATTER_A(msgA, dstA3, zeros)
        pB0, pB1 = _SCATTER_B(msgB, dstB3, zeros)
        h = _node_update(h, (pA0, pA1, pB0, pB1), lp['upd'], g, bln)
    return _gru(h, h_prev, params['gru'])


def kernel(x, edge_index, edge_attr, h_prev, params):
    return _run(x, edge_index, edge_attr, h_prev, params)


# speedup vs baseline: 1.0423x; 1.0025x over previous
import functools

import jax
import jax.numpy as jnp
from jax import lax
from jax.experimental import pallas as pl
from jax.experimental.pallas import tpu as pltpu
from jax.experimental.pallas import tpu_sc as plsc

N = 10000
E = 320000
H = 128

NW = 32
CH = 80
UNIT = CH * NW
EA = 80 * UNIT
EB = 45 * UNIT
NCA = EA // CH
NCB = EB // CH
SPWA = NCA // 16
SPWB = NCB // 16
NP = 10240
NPS = NP // 16

BN = 1000
BE = 4000

_f32 = jnp.float32


_bf16 = jnp.bfloat16
_i16 = jnp.int16
_i32 = jnp.int32


def _enc_body(x_ref, w1_ref, b1_ref, w2_ref, b2_ref, o_ref):
    t = jnp.dot(x_ref[...], w1_ref[...], preferred_element_type=_f32) + b1_ref[...]
    t = jnp.maximum(t, 0.0)
    y = jnp.dot(t, w2_ref[...], preferred_element_type=_f32) + b2_ref[...]
    o_ref[...] = y


def _encoder(x, enc):
    W1, b1, W2, b2 = enc
    n, d = x.shape
    grid = n // BN
    return pl.pallas_call(
        _enc_body,
        grid=(grid,),
        in_specs=[
            pl.BlockSpec((BN, d), lambda i: (i, 0)),
            pl.BlockSpec((d, H), lambda i: (0, 0)),
            pl.BlockSpec((1, H), lambda i: (0, 0)),
            pl.BlockSpec((H, H), lambda i: (0, 0)),
            pl.BlockSpec((1, H), lambda i: (0, 0)),
        ],
        out_specs=pl.BlockSpec((BN, H), lambda i: (i, 0)),
        out_shape=jax.ShapeDtypeStruct((n, H), _f32),
    )(x, W1, b1.reshape(1, H), W2, b2.reshape(1, H))


def _edge_body(xi_ref, xj_ref, ea_ref,
               w1i_ref, w1j_ref, w1e_ref, b1_ref, w2_ref, b2_ref,
               mw1_ref, mb1_ref, mw2_ref, mb2_ref,
               o_ref):
    t = (jnp.dot(xi_ref[...].astype(_bf16), w1i_ref[...], preferred_element_type=_f32)
         + jnp.dot(xj_ref[...].astype(_bf16), w1j_ref[...], preferred_element_type=_f32)
         + jnp.dot(ea_ref[...].astype(_bf16), w1e_ref[...], preferred_element_type=_f32)
         + b1_ref[...])
    t = jnp.maximum(t, 0.0).astype(_bf16)
    emb = jnp.dot(t, w2_ref[...], preferred_element_type=_f32) + b2_ref[...]
    t = jnp.maximum(
        jnp.dot(emb.astype(_bf16), mw1_ref[...], preferred_element_type=_f32)
        + mb1_ref[...], 0.0).astype(_bf16)
    o_ref[...] = jnp.dot(t, mw2_ref[...], preferred_element_type=_f32) + mb2_ref[...]


def _edge_mlp(xi, xj, ea, et, msg):
    W1, b1, W2, b2 = et
    MW1, mb1, MW2, mb2 = msg
    ed = ea.shape[1]
    e = xi.shape[0]
    be = e // 40
    grid = 40
    full = lambda i: (0, 0)
    return pl.pallas_call(
        _edge_body,
        grid=(grid,),
        in_specs=[
            pl.BlockSpec((be, H), lambda i: (i, 0)),
            pl.BlockSpec((be, H), lambda i: (i, 0)),
            pl.BlockSpec((be, ed), lambda i: (i, 0)),
            pl.BlockSpec((H, H), full),
            pl.BlockSpec((H, H), full),
            pl.BlockSpec((ed, H), full),
            pl.BlockSpec((1, H), full),
            pl.BlockSpec((H, H), full),
            pl.BlockSpec((1, H), full),
            pl.BlockSpec((H, H), full),
            pl.BlockSpec((1, H), full),
            pl.BlockSpec((H, H), full),
            pl.BlockSpec((1, H), full),
        ],
        out_specs=pl.BlockSpec((be, H), lambda i: (i, 0)),
        out_shape=jax.ShapeDtypeStruct((e, H), _f32),
    )(xi, xj, ea,
      W1[:H].astype(_bf16), W1[H:2 * H].astype(_bf16), W1[2 * H:].astype(_bf16),
      b1.reshape(1, H), W2.astype(_bf16), b2.reshape(1, H),
      MW1.astype(_bf16), mb1.reshape(1, H), MW2.astype(_bf16), mb2.reshape(1, H))


def _sc_mesh():
    return plsc.VectorSubcoreMesh(core_axis_name="c", subcore_axis_name="s")


def _make_gather(eh, cpw):
    def body(h_hbm, src3_hbm, dst3_hbm, xi_hbm, xj_hbm,
             idx_d, idx_s, rows_a, rows_b, sem_a, sem_b):
        cid = lax.axis_index("c")
        sid = lax.axis_index("s")
        wid = cid * 16 + sid
        c0 = wid * cpw
        pltpu.sync_copy(dst3_hbm.at[pl.ds(c0, cpw)], idx_d)
        pltpu.sync_copy(src3_hbm.at[pl.ds(c0, cpw)], idx_s)

        def one_pass(idx_v, out_hbm):
            pltpu.async_copy(h_hbm.at[idx_v.at[0, 0]], rows_a, sem_a)

            def pair(j, carry):
                ce = 2 * j
                pltpu.make_async_copy(h_hbm.at[idx_v.at[ce, 0]], rows_a,
                                      sem_a).wait()
                pltpu.async_copy(h_hbm.at[idx_v.at[ce + 1, 0]], rows_b, sem_b)
                pltpu.sync_copy(rows_a, out_hbm.at[pl.ds((c0 + ce) * CH, CH)])
                pltpu.make_async_copy(h_hbm.at[idx_v.at[ce + 1, 0]], rows_b,
                                      sem_b).wait()

                @pl.when(ce + 2 < cpw)
                def _():
                    pltpu.async_copy(h_hbm.at[idx_v.at[ce + 2, 0]], rows_a, sem_a)

                pltpu.sync_copy(rows_b, out_hbm.at[pl.ds((c0 + ce + 1) * CH, CH)])
                return carry

            lax.fori_loop(0, cpw // 2, pair, 0)
            if cpw % 2 == 1:
                pltpu.make_async_copy(h_hbm.at[idx_v.at[cpw - 1, 0]], rows_a,
                                      sem_a).wait()
                pltpu.sync_copy(rows_a, out_hbm.at[pl.ds((c0 + cpw - 1) * CH, CH)])

        one_pass(idx_d, xi_hbm)
        one_pass(idx_s, xj_hbm)

    return pl.kernel(
        body, mesh=_sc_mesh(),
        out_type=(jax.ShapeDtypeStruct((eh, H), _f32),
                  jax.ShapeDtypeStruct((eh, H), _f32)),
        scratch_types=[
            pltpu.VMEM((cpw, 1, CH), jnp.int32),
            pltpu.VMEM((cpw, 1, CH), jnp.int32),
            pltpu.VMEM((CH, H), _f32),
            pltpu.VMEM((CH, H), _f32),
            pltpu.SemaphoreType.DMA,
            pltpu.SemaphoreType.DMA,
        ],
    )


_GATHER_A = _make_gather(EA, NCA // NW)
_GATHER_B = _make_gather(EB, NCB // NW)


def _make_scatter(eh, spw):
    def body(msg_hbm, dst3_hbm, z_hbm, p0_hbm, p1_hbm,
             idx_all, rows_a, rows_b, acc_s, sem_a, sem_b):
        cid = lax.axis_index("c")
        sid = lax.axis_index("s")
        wid = cid * 16 + sid
        r0 = sid * NPS
        pltpu.sync_copy(z_hbm.at[pl.ds(r0, NPS)], acc_s.at[pl.ds(r0, NPS)])
        c0 = wid * spw
        pltpu.sync_copy(dst3_hbm.at[pl.ds(c0, spw)], idx_all)
        plsc.subcore_barrier()

        pltpu.async_copy(msg_hbm.at[pl.ds(c0 * CH, CH)], rows_a, sem_a)

        def pair(j, carry):
            ce = 2 * j
            pltpu.make_async_copy(msg_hbm.at[pl.ds((c0 + ce) * CH, CH)], rows_a,
                                  sem_a).wait()
            pltpu.async_copy(msg_hbm.at[pl.ds((c0 + ce + 1) * CH, CH)], rows_b,
                             sem_b)
            pltpu.sync_copy(rows_a, acc_s.at[idx_all.at[ce, 0]], add=True)
            pltpu.make_async_copy(msg_hbm.at[pl.ds((c0 + ce + 1) * CH, CH)],
                                  rows_b, sem_b).wait()

            @pl.when(ce + 2 < spw)
            def _():
                pltpu.async_copy(msg_hbm.at[pl.ds((c0 + ce + 2) * CH, CH)],
                                 rows_a, sem_a)

            pltpu.sync_copy(rows_b, acc_s.at[idx_all.at[ce + 1, 0]], add=True)
            return carry

        lax.fori_loop(0, spw // 2, pair, 0)
        if spw % 2 == 1:
            pltpu.make_async_copy(msg_hbm.at[pl.ds((c0 + spw - 1) * CH, CH)],
                                  rows_a, sem_a).wait()
            pltpu.sync_copy(rows_a, acc_s.at[idx_all.at[spw - 1, 0]], add=True)
        plsc.subcore_barrier()

        @pl.when(cid == 0)
        def _():
            pltpu.sync_copy(acc_s.at[pl.ds(r0, NPS)], p0_hbm.at[pl.ds(r0, NPS)])

        @pl.when(cid == 1)
        def _():
            pltpu.sync_copy(acc_s.at[pl.ds(r0, NPS)], p1_hbm.at[pl.ds(r0, NPS)])

    return pl.kernel(
        body, mesh=_sc_mesh(),
        out_type=(jax.ShapeDtypeStruct((NP, H), _f32),
                  jax.ShapeDtypeStruct((NP, H), _f32)),
        scratch_types=[
            pltpu.VMEM((spw, 1, CH), jnp.int32),
            pltpu.VMEM((CH, H), _f32),
            pltpu.VMEM((CH, H), _f32),
            pltpu.VMEM_SHARED((NP, H), _f32),
            pltpu.SemaphoreType.DMA,
            pltpu.SemaphoreType.DMA,
        ],
    )


_SCATTER_A = _make_scatter(EA, NCA // NW)
_SCATTER_B = _make_scatter(EB, NCB // NW)


def _node_body(h_ref, a0_ref, a1_ref, a2_ref, a3_ref, w1h_ref, w1a_ref, b1_ref,
               w2_ref, b2_ref, g_ref, bln_ref, o_ref):
    h = h_ref[...]
    aggr = (a0_ref[...] + a1_ref[...]) + (a2_ref[...] + a3_ref[...])
    t = (jnp.dot(h, w1h_ref[...], preferred_element_type=_f32)
         + jnp.dot(aggr, w1a_ref[...], preferred_element_type=_f32)
         + b1_ref[...])
    t = jnp.maximum(t, 0.0)
    upd = jnp.dot(t, w2_ref[...], preferred_element_type=_f32) + b2_ref[...]
    y = h + upd
    m = jnp.mean(y, axis=1, keepdims=True)
    c = y - m
    v = jnp.mean(c * c, axis=1, keepdims=True)
    o_ref[...] = c * jax.lax.rsqrt(v + 1e-5) * g_ref[...] + bln_ref[...]


def _node_update(h, parts, upd, g, bln):
    W1, b1, W2, b2 = upd
    grid = N // BN
    full = lambda i: (0, 0)
    return pl.pallas_call(
        _node_body,
        grid=(grid,),
        in_specs=[
            pl.BlockSpec((BN, H), lambda i: (i, 0)),
            pl.BlockSpec((BN, H), lambda i: (i, 0)),
            pl.BlockSpec((BN, H), lambda i: (i, 0)),
            pl.BlockSpec((BN, H), lambda i: (i, 0)),
            pl.BlockSpec((BN, H), lambda i: (i, 0)),
            pl.BlockSpec((H, H), full),
            pl.BlockSpec((H, H), full),
            pl.BlockSpec((1, H), full),
            pl.BlockSpec((H, H), full),
            pl.BlockSpec((1, H), full),
            pl.BlockSpec((1, H), full),
            pl.BlockSpec((1, H), full),
        ],
        out_specs=pl.BlockSpec((BN, H), lambda i: (i, 0)),
        out_shape=jax.ShapeDtypeStruct((N, H), _f32),
    )(h, parts[0], parts[1], parts[2], parts[3], W1[:H], W1[H:],
      b1.reshape(1, H), W2, b2.reshape(1, H), g.reshape(1, H),
      bln.reshape(1, H))


def _gru_body(h_ref, hp_ref, wih_ref, bih_ref, whh_ref, bhh_ref, o_ref):
    gi = jnp.dot(h_ref[...], wih_ref[...], preferred_element_type=_f32) + bih_ref[...]
    gh = jnp.dot(hp_ref[...], whh_ref[...], preferred_element_type=_f32) + bhh_ref[...]
    i_r, i_z, i_n = gi[:, :H], gi[:, H:2 * H], gi[:, 2 * H:]
    h_r, h_z, h_n = gh[:, :H], gh[:, H:2 * H], gh[:, 2 * H:]
    r = jax.nn.sigmoid(i_r + h_r)
    z = jax.nn.sigmoid(i_z + h_z)
    n_gate = jnp.tanh(i_n + r * h_n)
    o_ref[...] = (1.0 - z) * n_gate + z * hp_ref[...]


def _gru(h, h_prev, gru):
    W_ih, b_ih, W_hh, b_hh = gru
    grid = N // BN
    full = lambda i: (0, 0)
    return pl.pallas_call(
        _gru_body,
        grid=(grid,),
        in_specs=[
            pl.BlockSpec((BN, H), lambda i: (i, 0)),
            pl.BlockSpec((BN, H), lambda i: (i, 0)),
            pl.BlockSpec((H, 3 * H), full),
            pl.BlockSpec((1, 3 * H), full),
            pl.BlockSpec((H, 3 * H), full),
            pl.BlockSpec((1, 3 * H), full),
        ],
        out_specs=pl.BlockSpec((BN, H), lambda i: (i, 0)),
        out_shape=jax.ShapeDtypeStruct((N, H), _f32),
    )(h, h_prev, W_ih, b_ih.reshape(1, 3 * H), W_hh, b_hh.reshape(1, 3 * H))


@jax.jit
def _run(x, edge_index, edge_attr, h_prev, params):
    src = edge_index[0]
    dst = edge_index[1]
    srcA3 = src[:EA].reshape(NCA, 1, CH)
    srcB3 = src[EA:].reshape(NCB, 1, CH)
    dstA3 = dst[:EA].reshape(NCA, 1, CH)
    dstB3 = dst[EA:].reshape(NCB, 1, CH)
    eaA = edge_attr[:EA]
    eaB = edge_attr[EA:]
    zeros = jnp.zeros((NP, H), _f32)
    h = _encoder(x, params['enc'])
    g, bln = params['ln']
    for lp in params['layers']:
        xiA, xjA = _GATHER_A(h, srcA3, dstA3)
        xiB, xjB = _GATHER_B(h, srcB3, dstB3)
        msgA = _edge_mlp(xiA, xjA, eaA, lp['et'], lp['msg'])
        msgB = _edge_mlp(xiB, xjB, eaB, lp['et'], lp['msg'])
        pA0, pA1 = _SCATTER_A(msgA, dstA3, zeros)
        pB0, pB1 = _SCATTER_B(msgB, dstB3, zeros)
        h = _node_update(h, (pA0, pA1, pB0, pB1), lp['upd'], g, bln)
    return _gru(h, h_prev, params['gru'])


def kernel(x, edge_index, edge_attr, h_prev, params):
    return _run(x, edge_index, edge_attr, h_prev, params)
